# Initial kernel scaffold; baseline (speedup 1.0000x reference)
import jax
import jax.numpy as jnp
from jax.experimental import pallas as pl


def kernel(x, W, b):
    raise NotImplementedError("write your pallas kernel here")


# speedup vs baseline: 3.4380x; 3.4380x over previous
import functools

import jax
import jax.numpy as jnp
from jax import lax
from jax.experimental import pallas as pl
from jax.experimental.pallas import tpu as pltpu
from jax.experimental.pallas import tpu_sc as plsc

_DIMS = 64
_VOCAB = 100000
_TBLOCK = 4000


def _table_body(w_ref, b_ref, out_ref):
    out_ref[...] = w_ref[...].reshape(_DIMS, _TBLOCK).T + b_ref[...]


def _build_table(W, b):
    grid = (_VOCAB // _TBLOCK,)
    return pl.pallas_call(
        _table_body,
        grid=grid,
        in_specs=[
            pl.BlockSpec((_DIMS, 1, 1, _TBLOCK), lambda j: (0, j, 0, 0)),
            pl.BlockSpec((1, _DIMS), lambda j: (0, 0)),
        ],
        out_specs=pl.BlockSpec((_TBLOCK, _DIMS), lambda j: (j, 0)),
        out_shape=jax.ShapeDtypeStruct((_VOCAB, _DIMS), jnp.float32),
    )(W.reshape(_DIMS, _VOCAB // _TBLOCK, 1, _TBLOCK), b.reshape(1, _DIMS))


@functools.lru_cache(maxsize=None)
def _make_gather(B):
    info = plsc.get_sparse_core_info()
    NC, NS = info.num_cores, info.num_subcores
    NW = NC * NS
    K = 8
    CH = K * 128
    b_per_w = B // NW
    n_chunks = b_per_w // CH
    assert b_per_w % CH == 0
    mesh = plsc.VectorSubcoreMesh(core_axis_name="c", subcore_axis_name="s")

    @functools.partial(
        pl.kernel,
        mesh=mesh,
        out_type=jax.ShapeDtypeStruct((B, _DIMS), jnp.float32),
        scratch_types=[
            pltpu.VMEM((K, 128), jnp.int32),
            pltpu.VMEM((CH, _DIMS), jnp.float32),
            pltpu.SemaphoreType.DMA,
        ],
        compiler_params=pltpu.CompilerParams(use_tc_tiling_on_sc=False),
    )
    def gather_kernel(table_hbm, idx_hbm, out_hbm, idx_v, rows_v, sem):
        wid = lax.axis_index("s") * NC + lax.axis_index("c")
        row_base = wid * b_per_w
        idx_base = wid * (b_per_w // 128)

        def body(i, carry):
            off = row_base + i * CH
            pltpu.sync_copy(idx_hbm.at[pl.ds(idx_base + i * K, K)], idx_v)
            copies = [
                pltpu.async_copy(
                    table_hbm.at[idx_v.at[j]],
                    rows_v.at[pl.ds(j * 128, 128)],
                    sem,
                )
                for j in range(K)
            ]
            for cp in copies:
                cp.wait()
            pltpu.sync_copy(rows_v, out_hbm.at[pl.ds(off, CH)])
            return carry

        lax.fori_loop(0, n_chunks, body, 0)

    return gather_kernel


def kernel(x, W, b):
    table = _build_table(W, b)
    B = x.shape[0] * x.shape[1]
    idx2 = x.reshape(B // 128, 128)
    out_flat = _make_gather(B)(table, idx2)
    return out_flat.reshape(x.shape + (_DIMS,))


# speedup vs baseline: 3.8482x; 1.1193x over previous
import functools

import jax
import jax.numpy as jnp
from jax import lax
from jax.experimental import pallas as pl
from jax.experimental.pallas import tpu as pltpu
from jax.experimental.pallas import tpu_sc as plsc

_DIMS = 64
_VOCAB = 100000
_VPAD = 100096
_TBLOCK = 4352


def _table_body(w_ref, b_ref, out_ref):
    out_ref[...] = w_ref[...].T + b_ref[...]


def _build_table(W, b):
    grid = (_VPAD // _TBLOCK,)
    Wp = jnp.pad(W, ((0, 0), (0, _VPAD - _VOCAB)))
    return pl.pallas_call(
        _table_body,
        grid=grid,
        in_specs=[
            pl.BlockSpec((_DIMS, _TBLOCK), lambda j: (0, j)),
            pl.BlockSpec((1, _DIMS), lambda j: (0, 0)),
        ],
        out_specs=pl.BlockSpec((_TBLOCK, _DIMS), lambda j: (j, 0)),
        out_shape=jax.ShapeDtypeStruct((_VPAD, _DIMS), jnp.float32),
    )(Wp, b.reshape(1, _DIMS))


@functools.lru_cache(maxsize=None)
def _make_gather(batch, fields):
    info = plsc.get_sparse_core_info()
    NC, NS = info.num_cores, info.num_subcores
    NW = NC * NS
    CB = 64
    K = CB * fields // 128
    assert CB * fields % 128 == 0
    b_per_w = batch // NW
    n_chunks = b_per_w // CB
    assert b_per_w % CB == 0
    mesh = plsc.VectorSubcoreMesh(core_axis_name="c", subcore_axis_name="s")

    @functools.partial(
        pl.kernel,
        mesh=mesh,
        out_type=jax.ShapeDtypeStruct((batch, fields, _DIMS), jnp.float32),
        scratch_types=[
            pltpu.VMEM((K, 128), jnp.int32),
            pltpu.VMEM((CB * fields, _DIMS), jnp.float32),
            pltpu.SemaphoreType.DMA,
            pltpu.SemaphoreType.DMA,
        ],
        compiler_params=pltpu.CompilerParams(use_tc_tiling_on_sc=False),
    )
    def gather_kernel(table_hbm, idx_hbm, out_hbm, idx_v, rows_v, sem, sem2):
        wid = lax.axis_index("s") * NC + lax.axis_index("c")
        b_base = wid * b_per_w
        i_base = wid * (b_per_w * fields // 128)

        def body(i, carry):
            b0 = b_base + i * CB
            pltpu.sync_copy(idx_hbm.at[pl.ds(i_base + i * K, K)], idx_v)
            gathers = [
                pltpu.async_copy(
                    table_hbm.at[idx_v.at[j]],
                    rows_v.at[pl.ds(j * 128, 128)],
                    sem,
                )
                for j in range(K)
            ]
            for cp in gathers:
                cp.wait()
            writes = [
                pltpu.async_copy(
                    rows_v.at[pl.ds(r * fields, fields)],
                    out_hbm.at[b0 + r],
                    sem2,
                )
                for r in range(CB)
            ]
            for cp in writes:
                cp.wait()
            return carry

        lax.fori_loop(0, n_chunks, body, 0)

    return gather_kernel


def kernel(x, W, b):
    table = _build_table(W, b)
    batch, fields = x.shape
    idx2 = x.reshape(batch * fields // 128, 128)
    return _make_gather(batch, fields)(table, idx2)


# speedup vs baseline: 3.8577x; 1.0025x over previous
import functools

import jax
import jax.numpy as jnp
from jax import lax
from jax.experimental import pallas as pl
from jax.experimental.pallas import tpu as pltpu
from jax.experimental.pallas import tpu_sc as plsc

_DIMS = 64
_VOCAB = 100000
_VPAD = 100096
_TBLOCK = 4352


def _table_body(w_ref, b_ref, out_ref):
    out_ref[...] = w_ref[...].T + b_ref[...]


def _build_table(W, b):
    grid = (_VPAD // _TBLOCK,)
    Wp = jnp.pad(W, ((0, 0), (0, _VPAD - _VOCAB)))
    return pl.pallas_call(
        _table_body,
        grid=grid,
        in_specs=[
            pl.BlockSpec((_DIMS, _TBLOCK), lambda j: (0, j)),
            pl.BlockSpec((1, _DIMS), lambda j: (0, 0)),
        ],
        out_specs=pl.BlockSpec((_TBLOCK, _DIMS), lambda j: (j, 0)),
        out_shape=jax.ShapeDtypeStruct((_VPAD, _DIMS), jnp.float32),
    )(Wp, b.reshape(1, _DIMS))


@functools.lru_cache(maxsize=None)
def _make_gather(batch, fields):
    info = plsc.get_sparse_core_info()
    NC, NS = info.num_cores, info.num_subcores
    NW = NC * NS
    CB = 64
    K = CB * fields // 128
    assert CB * fields % 128 == 0
    b_per_w = batch // NW
    n_chunks = b_per_w // CB
    assert b_per_w % CB == 0
    mesh = plsc.VectorSubcoreMesh(core_axis_name="c", subcore_axis_name="s")

    @functools.partial(
        pl.kernel,
        mesh=mesh,
        out_type=jax.ShapeDtypeStruct((batch * fields, _DIMS), jnp.float32),
        scratch_types=[
            pltpu.VMEM((K, 128), jnp.int32),
            pltpu.VMEM((CB * fields, _DIMS), jnp.float32),
            pltpu.SemaphoreType.DMA,
            pltpu.SemaphoreType.DMA,
        ],
        compiler_params=pltpu.CompilerParams(use_tc_tiling_on_sc=False),
    )
    def gather_kernel(table_hbm, idx_hbm, out_hbm, idx_v, rows_v, sem, sem2):
        wid = lax.axis_index("s") * NC + lax.axis_index("c")
        b_base = wid * b_per_w
        i_base = wid * (b_per_w * fields // 128)

        def body(i, carry):
            b0 = b_base + i * CB
            pltpu.sync_copy(idx_hbm.at[pl.ds(i_base + i * K, K)], idx_v)
            gathers = [
                pltpu.async_copy(
                    table_hbm.at[idx_v.at[j]],
                    rows_v.at[pl.ds(j * 128, 128)],
                    sem,
                )
                for j in range(K)
            ]
            for cp in gathers:
                cp.wait()
            pltpu.async_copy(
                rows_v,
                out_hbm.at[pl.ds(b0 * fields, CB * fields)],
                sem2,
            ).wait()
            return carry

        lax.fori_loop(0, n_chunks, body, 0)

    return gather_kernel


def kernel(x, W, b):
    table = _build_table(W, b)
    batch, fields = x.shape
    idx2 = x.reshape(batch * fields // 128, 128)
    out2 = _make_gather(batch, fields)(table, idx2)
    return out2.reshape(batch, fields, _DIMS)


# speedup vs baseline: 4.2175x; 1.0933x over previous
import functools

import jax
import jax.numpy as jnp
from jax import lax
from jax.experimental import pallas as pl
from jax.experimental.pallas import tpu as pltpu
from jax.experimental.pallas import tpu_sc as plsc

_DIMS = 64
_VOCAB = 100000
_VPAD = 100096
_TBLOCK = 4352


def _table_body(w_ref, b_ref, out_ref):
    out_ref[:, 0:_DIMS] = w_ref[...].T + b_ref[...]
    out_ref[:, _DIMS:128] = jnp.zeros((out_ref.shape[0], 128 - _DIMS),
                                      jnp.float32)


def _build_table(W, b):
    grid = (_VPAD // _TBLOCK,)
    Wp = jnp.pad(W, ((0, 0), (0, _VPAD - _VOCAB)))
    return pl.pallas_call(
        _table_body,
        grid=grid,
        in_specs=[
            pl.BlockSpec((_DIMS, _TBLOCK), lambda j: (0, j)),
            pl.BlockSpec((1, _DIMS), lambda j: (0, 0)),
        ],
        out_specs=pl.BlockSpec((_TBLOCK, 128), lambda j: (j, 0)),
        out_shape=jax.ShapeDtypeStruct((_VPAD, 128), jnp.float32),
    )(Wp, b.reshape(1, _DIMS))


@functools.lru_cache(maxsize=None)
def _make_gather(batch, fields):
    info = plsc.get_sparse_core_info()
    NC, NS = info.num_cores, info.num_subcores
    NW = NC * NS
    CB = 64
    K = CB * fields // 128
    assert CB * fields % 128 == 0
    b_per_w = batch // NW
    n_chunks = b_per_w // CB
    assert b_per_w % CB == 0
    mesh = plsc.VectorSubcoreMesh(core_axis_name="c", subcore_axis_name="s")

    @functools.partial(
        pl.kernel,
        mesh=mesh,
        out_type=jax.ShapeDtypeStruct((batch * fields, _DIMS), jnp.float32),
        scratch_types=[
            pltpu.VMEM((K, 128), jnp.int32),
            pltpu.VMEM((CB * fields, _DIMS), jnp.float32),
            pltpu.SemaphoreType.DMA,
            pltpu.SemaphoreType.DMA,
        ],
        compiler_params=pltpu.CompilerParams(use_tc_tiling_on_sc=False),
    )
    def gather_kernel(table_hbm, idx_hbm, out_hbm, idx_v, rows_v, sem, sem2):
        wid = lax.axis_index("s") * NC + lax.axis_index("c")
        b_base = wid * b_per_w
        i_base = wid * (b_per_w * fields // 128)

        def body(i, carry):
            b0 = b_base + i * CB
            pltpu.sync_copy(idx_hbm.at[pl.ds(i_base + i * K, K)], idx_v)
            gathers = [
                pltpu.async_copy(
                    table_hbm.at[idx_v.at[j]],
                    rows_v.at[pl.ds(j * 128, 128)],
                    sem,
                )
                for j in range(K)
            ]
            for cp in gathers:
                cp.wait()
            pltpu.async_copy(
                rows_v,
                out_hbm.at[pl.ds(b0 * fields, CB * fields)],
                sem2,
            ).wait()
            return carry

        lax.fori_loop(0, n_chunks, body, 0)

    return gather_kernel


def kernel(x, W, b):
    table = _build_table(W, b).reshape(2 * _VPAD, _DIMS)
    batch, fields = x.shape
    idx2 = (x * 2).reshape(batch * fields // 128, 128)
    out2 = _make_gather(batch, fields)(table, idx2)
    return out2.reshape(batch, fields, _DIMS)


# speedup vs baseline: 6.5803x; 1.5602x over previous
import functools

import jax
import jax.numpy as jnp
from jax import lax
from jax.experimental import pallas as pl
from jax.experimental.pallas import tpu as pltpu
from jax.experimental.pallas import tpu_sc as plsc

_DIMS = 64
_VOCAB = 100000
_VPAD = 100096
_TBLOCK = 4352


def _table_body(w_ref, b_ref, out_ref):
    out_ref[:, 0:_DIMS] = w_ref[...].T + b_ref[...]
    out_ref[:, _DIMS:128] = jnp.zeros((out_ref.shape[0], 128 - _DIMS),
                                      jnp.float32)


def _build_table(W, b):
    grid = (_VPAD // _TBLOCK,)
    Wp = jnp.pad(W, ((0, 0), (0, _VPAD - _VOCAB)))
    return pl.pallas_call(
        _table_body,
        grid=grid,
        in_specs=[
            pl.BlockSpec((_DIMS, _TBLOCK), lambda j: (0, j)),
            pl.BlockSpec((1, _DIMS), lambda j: (0, 0)),
        ],
        out_specs=pl.BlockSpec((_TBLOCK, 128), lambda j: (j, 0)),
        out_shape=jax.ShapeDtypeStruct((_VPAD, 128), jnp.float32),
    )(Wp, b.reshape(1, _DIMS))


@functools.lru_cache(maxsize=None)
def _make_gather(batch, fields):
    info = plsc.get_sparse_core_info()
    NC, NS = info.num_cores, info.num_subcores
    NW = NC * NS
    CB = 64
    K = CB * fields // 128
    assert CB * fields % 128 == 0
    b_per_w = batch // NW
    n_chunks = b_per_w // CB
    assert b_per_w % CB == 0
    mesh = plsc.VectorSubcoreMesh(core_axis_name="c", subcore_axis_name="s")

    @functools.partial(
        pl.kernel,
        mesh=mesh,
        out_type=jax.ShapeDtypeStruct((batch * fields, _DIMS), jnp.float32),
        scratch_types=[
            pltpu.VMEM((K, 128), jnp.int32),
            pltpu.VMEM((CB * fields, _DIMS), jnp.float32),
            pltpu.SemaphoreType.DMA,
            pltpu.SemaphoreType.DMA,
        ],
        compiler_params=pltpu.CompilerParams(use_tc_tiling_on_sc=False),
    )
    def gather_kernel(table_hbm, idx_hbm, out_hbm, idx_v, rows_v, sem, sem2):
        wid = lax.axis_index("s") * NC + lax.axis_index("c")
        b_base = wid * b_per_w
        i_base = wid * (b_per_w * fields // 128)

        def body(i, carry):
            b0 = b_base + i * CB
            pltpu.sync_copy(idx_hbm.at[pl.ds(i_base + i * K, K)], idx_v)
            gathers = [
                pltpu.async_copy(
                    table_hbm.at[idx_v.at[j]],
                    rows_v.at[pl.ds(j * 128, 128)],
                    sem,
                )
                for j in range(K)
            ]
            for cp in gathers:
                cp.wait()
            pltpu.async_copy(
                rows_v,
                out_hbm.at[pl.ds(b0 * fields, CB * fields)],
                sem2,
            ).wait()
            return carry

        lax.fori_loop(0, n_chunks, body, 0)

    return gather_kernel


_TRB = 256


def _tr_body(in_ref, out_ref):
    for j in range(13):
        pair = in_ref[j::13, :]
        out_ref[2 * j] = pair[:, 0:_DIMS].T
        out_ref[2 * j + 1] = pair[:, _DIMS:128].T


def _transpose_out(flat, batch, fields):
    grid = (batch // _TRB,)
    return pl.pallas_call(
        _tr_body,
        grid=grid,
        in_specs=[pl.BlockSpec((_TRB * fields // 2, 128), lambda j: (j, 0))],
        out_specs=pl.BlockSpec((fields, _DIMS, _TRB), lambda j: (0, 0, j)),
        out_shape=jax.ShapeDtypeStruct((fields, _DIMS, batch), jnp.float32),
    )(flat)


def kernel(x, W, b):
    table = _build_table(W, b).reshape(2 * _VPAD, _DIMS)
    batch, fields = x.shape
    idx2 = (x * 2).reshape(batch * fields // 128, 128)
    out2 = _make_gather(batch, fields)(table, idx2)
    out_t = _transpose_out(
        out2.reshape(batch * fields // 2, 128), batch, fields
    )
    return jnp.transpose(out_t, (2, 0, 1))
